# baseline (speedup 1.0000x reference)
import functools

import jax
import jax.numpy as jnp
from jax.experimental import pallas as pl
from jax.experimental.pallas import tpu as pltpu

_P = 0.0005
_FMIN = float(jnp.finfo(jnp.float32).min)


def _desc_rank_row(w):
    r, n = w.shape
    wd = w[:, :, None]
    we = w[:, None, :]
    d_idx = jax.lax.broadcasted_iota(jnp.int32, (r, n, n), 1)
    e_idx = jax.lax.broadcasted_iota(jnp.int32, (r, n, n), 2)
    beats = (we > wd) | ((we == wd) & (e_idx < d_idx))
    return jnp.sum(beats.astype(jnp.int32), axis=2)


def _desc_rank_col(w):
    n, b = w.shape
    wd = w[:, None, :]
    we = w[None, :, :]
    d_idx = jax.lax.broadcasted_iota(jnp.int32, (n, n, b), 0)
    e_idx = jax.lax.broadcasted_iota(jnp.int32, (n, n, b), 1)
    beats = (we > wd) | ((we == wd) & (e_idx < d_idx))
    return jnp.sum(beats.astype(jnp.int32), axis=1)


def _fused_body(feat_ref, x_ref, xf_ref, wt_ref, b_ref, out_ref,
                m_ref, s_ref, mf_ref, sf_ref, *, n_blocks, blk, c, k, b_sz):
    i = pl.program_id(0)
    j = (i + 1) % n_blocks

    @pl.when(i == 0)
    def _init():
        neg = jnp.full((b_sz, blk), _FMIN, jnp.float32)
        zero = jnp.zeros((b_sz, blk), jnp.float32)
        m_ref[...] = neg
        s_ref[...] = zero
        mf_ref[...] = neg
        sf_ref[...] = zero

    def _acc(x, m_r, s_r):
        m_old = m_r[...]
        m_new = jnp.maximum(m_old, x)
        s_r[...] = s_r[...] * jnp.exp(m_old - m_new) + jnp.exp(x - m_new)
        m_r[...] = m_new

    rem_w = c - (n_blocks - 1) * blk
    if rem_w == blk:
        _acc(x_ref[...], m_ref, s_ref)
        _acc(xf_ref[...], mf_ref, sf_ref)
    else:
        @pl.when(j != n_blocks - 1)
        def _full():
            _acc(x_ref[...], m_ref, s_ref)
            _acc(xf_ref[...], mf_ref, sf_ref)

        @pl.when(j == n_blocks - 1)
        def _ragged():
            valid = (jax.lax.broadcasted_iota(jnp.int32, (b_sz, blk), 1)
                     < rem_w)
            _acc(jnp.where(valid, x_ref[...], _FMIN), m_ref, s_ref)
            _acc(jnp.where(valid, xf_ref[...], _FMIN), mf_ref, sf_ref)

    feat = feat_ref[...]
    dims = (((1,), (0,)), ((), ()))
    y = jax.lax.dot_general(feat, wt_ref[...], dims,
                            preferred_element_type=jnp.float32)
    out_ref[...] = y + b_ref[...][None, :]

    @pl.when(i == n_blocks - 1)
    def _last():
        def _finish(m_r, s_r, x0):
            m_vec = m_r[...]
            m_row = jnp.max(m_vec, axis=1, keepdims=True)
            s_row = jnp.sum(s_r[...] * jnp.exp(m_vec - m_row),
                            axis=1, keepdims=True)
            return jnp.exp(x0 - m_row) / s_row

        h = _finish(m_ref, s_ref, x_ref[:, 0:1])
        hf = _finish(mf_ref, sf_ref, xf_ref[:, 0:1])

        eye = (jax.lax.broadcasted_iota(jnp.int32, (b_sz, b_sz), 0)
               == jax.lax.broadcasted_iota(jnp.int32, (b_sz, b_sz), 1))

        def _trow(col):
            return jnp.sum(jnp.where(eye, col, 0), axis=0, keepdims=True)

        def _tcol(row):
            return jnp.sum(jnp.where(eye, row, 0), axis=1, keepdims=True)

        rank_h = _desc_rank_row(_trow(h))
        rank_hf = _desc_rank_row(_trow(hf))
        pair = rank_hf == _tcol(rank_h)

        wt16 = wt_ref[:, 0:b_sz]
        rd = _desc_rank_col(wt16)
        w_src = jnp.sum(jnp.where(pair[None, :, :], wt16[:, None, :], 0.0),
                        axis=2)
        r_src = jnp.sum(jnp.where(pair[None, :, :], rd[:, None, :], 0),
                        axis=2)
        take = r_src[None, :, :] == rd[:, None, :]
        newval = jnp.sum(jnp.where(take, w_src[None, :, :], 0.0), axis=1)
        wt16_mod = jnp.where(rd < k, newval, wt16)

        b16 = b_ref[0:b_sz][None, :]
        b16_mod = _trow(jnp.sum(jnp.where(pair, b16, 0.0),
                                axis=1, keepdims=True))

        y16 = jax.lax.dot_general(feat, wt16_mod, dims,
                                  preferred_element_type=jnp.float32)
        out_ref[:, 0:b_sz] = y16 + b16_mod


def kernel(features, features_f, output, output_f, weight_matrix, bias):
    del features_f
    b_sz, d = features.shape
    c = weight_matrix.shape[0]
    k = int(round(c * _P))
    blk = 32768
    n_blocks = pl.cdiv(c, blk)
    wt = weight_matrix.T

    shift = lambda i: (i + 1) % n_blocks
    body = functools.partial(_fused_body, n_blocks=n_blocks, blk=blk, c=c,
                             k=k, b_sz=b_sz)
    return pl.pallas_call(
        body,
        grid=(n_blocks,),
        in_specs=[
            pl.BlockSpec((b_sz, d), lambda i: (0, 0)),
            pl.BlockSpec((b_sz, blk), lambda i: (0, shift(i))),
            pl.BlockSpec((b_sz, blk), lambda i: (0, shift(i))),
            pl.BlockSpec((d, blk), lambda i: (0, shift(i))),
            pl.BlockSpec((blk,), lambda i: (shift(i),)),
        ],
        out_specs=pl.BlockSpec((b_sz, blk), lambda i: (0, shift(i))),
        out_shape=jax.ShapeDtypeStruct((b_sz, c), jnp.float32),
        scratch_shapes=[pltpu.VMEM((b_sz, blk), jnp.float32)] * 4,
        compiler_params=pltpu.CompilerParams(
            dimension_semantics=("arbitrary",)),
    )(features, output, output_f, wt, bias)
